# baseline (speedup 1.0000x reference)
import jax
import jax.numpy as jnp
from jax import lax
from jax.experimental import pallas as pl
from jax.experimental.pallas import tpu as pltpu
from jax.experimental.pallas import tpu_sc as plsc

BATCH = 16384
EMB_DIM = 32
N_USER_ROWS = 1000000
N_ITEM_ROWS = 100000
NUM_CORES = 2
NUM_SUBCORES = 16
NUM_WORKERS = NUM_CORES * NUM_SUBCORES
B_PER_W = BATCH // NUM_WORKERS
LANES = 16
BLOCKS = B_PER_W // LANES
SLAB = EMB_DIM * B_PER_W


def _dot_kernel(uidx_hbm, iidx_hbm, uflat_hbm, iflat_hbm, out_hbm,
                uidx_v, iidx_v, ubuf, ibuf, out_v, sem_s, sem_u, sem_i):
    wid = lax.axis_index("s") * NUM_CORES + lax.axis_index("c")

    cs_u = pltpu.async_copy(uidx_hbm.at[wid], uidx_v, sem_s)
    cs_i = pltpu.async_copy(iidx_hbm.at[wid], iidx_v, sem_s)
    cs_u.wait()
    cs_i.wait()

    cu = pltpu.async_copy(uflat_hbm.at[uidx_v], ubuf, sem_u)
    ci = pltpu.async_copy(iflat_hbm.at[iidx_v], ibuf, sem_i)
    cu.wait()
    ci.wait()

    @pl.loop(0, BLOCKS)
    def _(j):
        acc = jnp.zeros((LANES,), jnp.float32)
        for d in range(EMB_DIM):
            acc = acc + (ubuf[pl.ds(d * B_PER_W + j * LANES, LANES)]
                         * ibuf[pl.ds(d * B_PER_W + j * LANES, LANES)])
        out_v[pl.ds(j * LANES, LANES)] = acc

    pltpu.sync_copy(out_v, out_hbm.at[pl.ds(wid * B_PER_W, B_PER_W)])


@jax.jit
def kernel(user, item, user_emb, item_emb):
    user = user.astype(jnp.int32)
    item = item.astype(jnp.int32)
    dim_off = lax.iota(jnp.int32, EMB_DIM)
    uidx = (user.reshape(NUM_WORKERS, 1, B_PER_W)
            + (dim_off * N_USER_ROWS)[None, :, None]
            ).reshape(NUM_WORKERS, SLAB)
    iidx = (item.reshape(NUM_WORKERS, 1, B_PER_W)
            + (dim_off * N_ITEM_ROWS)[None, :, None]
            ).reshape(NUM_WORKERS, SLAB)
    uflat = user_emb.T.reshape(-1)
    iflat = item_emb.T.reshape(-1)

    mesh = plsc.VectorSubcoreMesh(core_axis_name="c", subcore_axis_name="s")
    run = pl.kernel(
        _dot_kernel,
        out_type=jax.ShapeDtypeStruct((BATCH,), jnp.float32),
        mesh=mesh,
        compiler_params=pltpu.CompilerParams(needs_layout_passes=False),
        scratch_types=[
            pltpu.VMEM((SLAB,), jnp.int32),
            pltpu.VMEM((SLAB,), jnp.int32),
            pltpu.VMEM((SLAB,), jnp.float32),
            pltpu.VMEM((SLAB,), jnp.float32),
            pltpu.VMEM((B_PER_W,), jnp.float32),
            pltpu.SemaphoreType.DMA,
            pltpu.SemaphoreType.DMA,
            pltpu.SemaphoreType.DMA,
        ],
    )
    return run(uidx, iidx, uflat, iflat)


# speedup vs baseline: 3.2949x; 3.2949x over previous
import jax
import jax.numpy as jnp
from jax import lax
from jax.experimental import pallas as pl
from jax.experimental.pallas import tpu as pltpu
from jax.experimental.pallas import tpu_sc as plsc

BATCH = 16384
EMB_DIM = 32
NUM_CORES = 2
NUM_SUBCORES = 16
NUM_WORKERS = NUM_CORES * NUM_SUBCORES
B_PER_W = BATCH // NUM_WORKERS
LANES = 16
BLOCKS = B_PER_W // LANES


def _dot_kernel(*refs):
    uidx_hbm = refs[0]
    iidx_hbm = refs[1]
    ucols_hbm = refs[2:2 + EMB_DIM]
    icols_hbm = refs[2 + EMB_DIM:2 + 2 * EMB_DIM]
    out_hbm = refs[2 + 2 * EMB_DIM]
    uidx_v = refs[3 + 2 * EMB_DIM]
    iidx_v = refs[4 + 2 * EMB_DIM]
    ubufs = refs[5 + 2 * EMB_DIM:5 + 3 * EMB_DIM]
    ibufs = refs[5 + 3 * EMB_DIM:5 + 4 * EMB_DIM]
    out_v = refs[5 + 4 * EMB_DIM]
    sem_s, sem_u, sem_i = refs[6 + 4 * EMB_DIM:9 + 4 * EMB_DIM]

    wid = lax.axis_index("s") * NUM_CORES + lax.axis_index("c")
    base = wid * B_PER_W

    cs_u = pltpu.async_copy(uidx_hbm.at[pl.ds(base, B_PER_W)], uidx_v, sem_s)
    cs_i = pltpu.async_copy(iidx_hbm.at[pl.ds(base, B_PER_W)], iidx_v, sem_s)
    cs_u.wait()
    cs_i.wait()

    copies = []
    for d in range(EMB_DIM):
        copies.append(
            pltpu.async_copy(ucols_hbm[d].at[uidx_v], ubufs[d], sem_u))
        copies.append(
            pltpu.async_copy(icols_hbm[d].at[iidx_v], ibufs[d], sem_i))
    for cp in copies:
        cp.wait()

    @pl.loop(0, BLOCKS)
    def _(j):
        acc = jnp.zeros((LANES,), jnp.float32)
        for d in range(EMB_DIM):
            acc = acc + (ubufs[d][pl.ds(j * LANES, LANES)]
                         * ibufs[d][pl.ds(j * LANES, LANES)])
        out_v[pl.ds(j * LANES, LANES)] = acc

    pltpu.sync_copy(out_v, out_hbm.at[pl.ds(base, B_PER_W)])


@jax.jit
def kernel(user, item, user_emb, item_emb):
    user = user.astype(jnp.int32)
    item = item.astype(jnp.int32)
    ucols = [user_emb[:, d] for d in range(EMB_DIM)]
    icols = [item_emb[:, d] for d in range(EMB_DIM)]

    mesh = plsc.VectorSubcoreMesh(core_axis_name="c", subcore_axis_name="s")
    scratch = [
        pltpu.VMEM((B_PER_W,), jnp.int32),
        pltpu.VMEM((B_PER_W,), jnp.int32),
    ]
    scratch += [pltpu.VMEM((B_PER_W,), jnp.float32)
                for _ in range(2 * EMB_DIM)]
    scratch += [
        pltpu.VMEM((B_PER_W,), jnp.float32),
        pltpu.SemaphoreType.DMA,
        pltpu.SemaphoreType.DMA,
        pltpu.SemaphoreType.DMA,
    ]
    run = pl.kernel(
        _dot_kernel,
        out_type=jax.ShapeDtypeStruct((BATCH,), jnp.float32),
        mesh=mesh,
        compiler_params=pltpu.CompilerParams(needs_layout_passes=False),
        scratch_types=scratch,
    )
    return run(user, item, *ucols, *icols)


# speedup vs baseline: 14.1278x; 4.2878x over previous
import jax
import jax.numpy as jnp
from jax import lax
from jax.experimental import pallas as pl
from jax.experimental.pallas import tpu as pltpu
from jax.experimental.pallas import tpu_sc as plsc

BATCH = 16384
EMB_DIM = 32
N_USER_ROWS = 1000000
N_ITEM_ROWS = 100000
NUM_CORES = 2
NUM_SUBCORES = 16
NUM_WORKERS = NUM_CORES * NUM_SUBCORES
B_PER_W = BATCH // NUM_WORKERS
LANES = 16
BLOCKS = B_PER_W // LANES
SLAB = EMB_DIM * B_PER_W
DGROUPS = EMB_DIM // 8

U_EFF = 999936
U_APP = N_USER_ROWS - U_EFF
U_SPAN = 31232
U_LSPAN = U_EFF - 31 * U_SPAN
U_CHUNK = 7808
U_LCHUNK = 7936
U_APP_OFF = EMB_DIM * U_EFF

I_EFF = 99968
I_APP = N_ITEM_ROWS - I_EFF
I_SPAN = 3072
I_LSPAN = I_EFF - 31 * I_SPAN
I_APP_OFF = EMB_DIM * I_EFF


def _relayout_kernel(uT_hbm, iT_hbm, uapp_hbm, iapp_hbm,
                     uflat_hbm, iflat_hbm,
                     slab, isl, uapp_v, iapp_v, sem_in, sem_out):
    wid = lax.axis_index("s") * NUM_CORES + lax.axis_index("c")
    is_last = wid >= NUM_WORKERS - 1

    def move(src_hbm, dst_hbm, n_eff, buf, dg, off, ln, dst_off):
        pltpu.async_copy(
            src_hbm.at[pl.ds(dg * 8, 8), pl.ds(off, ln)],
            buf.at[pl.ds(0, 8), pl.ds(0, ln)], sem_in).wait()
        cps = []
        for dd in range(8):
            d = dg * 8 + dd
            cps.append(pltpu.async_copy(
                buf.at[dd, pl.ds(0, ln)],
                dst_hbm.at[pl.ds(d * n_eff + dst_off, ln)], sem_out))
        for cp in cps:
            cp.wait()

    for dg in range(DGROUPS):
        for k in range(4):
            @pl.when(jnp.logical_not(is_last))
            def _(dg=dg, k=k):
                off = wid * U_SPAN + k * U_CHUNK
                move(uT_hbm, uflat_hbm, U_EFF, slab, dg, off, U_CHUNK, off)

            @pl.when(is_last)
            def _(dg=dg, k=k):
                off = 31 * U_SPAN + k * U_LCHUNK
                move(uT_hbm, uflat_hbm, U_EFF, slab, dg, off, U_LCHUNK, off)

    for dg in range(DGROUPS):
        @pl.when(jnp.logical_not(is_last))
        def _(dg=dg):
            off = wid * I_SPAN
            move(iT_hbm, iflat_hbm, I_EFF, isl, dg, off, I_SPAN, off)

        @pl.when(is_last)
        def _(dg=dg):
            off = 31 * I_SPAN
            move(iT_hbm, iflat_hbm, I_EFF, isl, dg, off, I_LSPAN, off)

    @pl.when(wid == 0)
    def _():
        cu = pltpu.async_copy(uapp_hbm, uapp_v, sem_in)
        ci = pltpu.async_copy(iapp_hbm, iapp_v, sem_in)
        cu.wait()
        ci.wait()
        cps = []
        for d in range(EMB_DIM):
            cps.append(pltpu.async_copy(
                uapp_v.at[d],
                uflat_hbm.at[pl.ds(U_APP_OFF + d * U_APP, U_APP)], sem_out))
            cps.append(pltpu.async_copy(
                iapp_v.at[d],
                iflat_hbm.at[pl.ds(I_APP_OFF + d * I_APP, I_APP)], sem_out))
        for cp in cps:
            cp.wait()


def _dot_kernel(uidx_hbm, iidx_hbm, uflat_hbm, iflat_hbm, out_hbm,
                uidx_v, iidx_v, ubuf, ibuf, out_v, sem_s, sem_u, sem_i):
    wid = lax.axis_index("s") * NUM_CORES + lax.axis_index("c")

    cs_u = pltpu.async_copy(uidx_hbm.at[wid], uidx_v, sem_s)
    cs_i = pltpu.async_copy(iidx_hbm.at[wid], iidx_v, sem_s)
    cs_u.wait()
    cs_i.wait()

    cu = pltpu.async_copy(uflat_hbm.at[uidx_v], ubuf, sem_u)
    ci = pltpu.async_copy(iflat_hbm.at[iidx_v], ibuf, sem_i)
    cu.wait()
    ci.wait()

    @pl.loop(0, BLOCKS)
    def _(j):
        acc = jnp.zeros((LANES,), jnp.float32)
        for d in range(EMB_DIM):
            acc = acc + (ubuf[pl.ds(d * B_PER_W + j * LANES, LANES)]
                         * ibuf[pl.ds(d * B_PER_W + j * LANES, LANES)])
        out_v[pl.ds(j * LANES, LANES)] = acc

    pltpu.sync_copy(out_v, out_hbm.at[pl.ds(wid * B_PER_W, B_PER_W)])


@jax.jit
def kernel(user, item, user_emb, item_emb):
    user = user.astype(jnp.int32)
    item = item.astype(jnp.int32)
    dim_off = lax.iota(jnp.int32, EMB_DIM)

    u_b = user.reshape(NUM_WORKERS, 1, B_PER_W)
    i_b = item.reshape(NUM_WORKERS, 1, B_PER_W)
    d_b = dim_off[None, :, None]
    uidx = jnp.where(u_b < U_EFF,
                     d_b * U_EFF + u_b,
                     U_APP_OFF + d_b * U_APP + (u_b - U_EFF)
                     ).reshape(NUM_WORKERS, SLAB)
    iidx = jnp.where(i_b < I_EFF,
                     d_b * I_EFF + i_b,
                     I_APP_OFF + d_b * I_APP + (i_b - I_EFF)
                     ).reshape(NUM_WORKERS, SLAB)

    uapp = user_emb[U_EFF:, :].T
    iapp = item_emb[I_EFF:, :].T

    mesh = plsc.VectorSubcoreMesh(core_axis_name="c", subcore_axis_name="s")
    relayout = pl.kernel(
        _relayout_kernel,
        out_type=(jax.ShapeDtypeStruct((EMB_DIM * N_USER_ROWS,), jnp.float32),
                  jax.ShapeDtypeStruct((EMB_DIM * N_ITEM_ROWS,), jnp.float32)),
        mesh=mesh,
        compiler_params=pltpu.CompilerParams(needs_layout_passes=False),
        scratch_types=[
            pltpu.VMEM((8, U_LCHUNK), jnp.float32),
            pltpu.VMEM((8, I_LSPAN), jnp.float32),
            pltpu.VMEM((EMB_DIM, U_APP), jnp.float32),
            pltpu.VMEM((EMB_DIM, I_APP), jnp.float32),
            pltpu.SemaphoreType.DMA,
            pltpu.SemaphoreType.DMA,
        ],
    )
    uflat, iflat = relayout(user_emb.T, item_emb.T, uapp, iapp)

    gather = pl.kernel(
        _dot_kernel,
        out_type=jax.ShapeDtypeStruct((BATCH,), jnp.float32),
        mesh=mesh,
        compiler_params=pltpu.CompilerParams(needs_layout_passes=False),
        scratch_types=[
            pltpu.VMEM((SLAB,), jnp.int32),
            pltpu.VMEM((SLAB,), jnp.int32),
            pltpu.VMEM((SLAB,), jnp.float32),
            pltpu.VMEM((SLAB,), jnp.float32),
            pltpu.VMEM((B_PER_W,), jnp.float32),
            pltpu.SemaphoreType.DMA,
            pltpu.SemaphoreType.DMA,
            pltpu.SemaphoreType.DMA,
        ],
    )
    return gather(uidx, iidx, uflat, iflat)


# speedup vs baseline: 14.6333x; 1.0358x over previous
import jax
import jax.numpy as jnp
from jax import lax
from jax.experimental import pallas as pl
from jax.experimental.pallas import tpu as pltpu
from jax.experimental.pallas import tpu_sc as plsc

BATCH = 16384
EMB_DIM = 32
N_USER_ROWS = 1000000
N_ITEM_ROWS = 100000
NUM_CORES = 2
NUM_SUBCORES = 16
NUM_WORKERS = NUM_CORES * NUM_SUBCORES
B_PER_W = BATCH // NUM_WORKERS
LANES = 16
BLOCKS = B_PER_W // LANES
SLAB = EMB_DIM * B_PER_W
DGROUPS = EMB_DIM // 8

U_EFF = 999936
U_APP = N_USER_ROWS - U_EFF
U_SPAN = 31232
U_REM_OFF = NUM_WORKERS * U_SPAN
U_REM = U_EFF - U_REM_OFF
SLAB_W = 3968
U_CHUNKS = [3968] * 7 + [3456]
U_APP_OFF = EMB_DIM * U_EFF

I_EFF = 99968
I_APP = N_ITEM_ROWS - I_EFF
I_SPAN = 3072
I_REM_OFF = NUM_WORKERS * I_SPAN
I_REM = I_EFF - I_REM_OFF
I_CHUNKS = [3072]
I_APP_OFF = EMB_DIM * I_EFF


def _relayout_kernel(uT_hbm, iT_hbm, uapp_hbm, iapp_hbm,
                     uflat_hbm, iflat_hbm,
                     slab0, slab1, uapp_v, iapp_v, sem_in, sem_out):
    wid = lax.axis_index("s") * NUM_CORES + lax.axis_index("c")
    is_last = wid >= NUM_WORKERS - 1
    slabs = (slab0, slab1)

    steps = []
    for dg in range(DGROUPS):
        coff = 0
        for ln in U_CHUNKS:
            steps.append(("u", dg, coff, ln))
            coff += ln
        steps.append(("i", dg, 0, I_SPAN))

    def params(st, ubase, ibase):
        tab, dg, coff, ln = st
        if tab == "u":
            return uT_hbm, uflat_hbm, U_EFF, dg, ubase + coff, ln
        return iT_hbm, iflat_hbm, I_EFF, dg, ibase + coff, ln

    def issue_in(st, ubase, ibase, buf):
        src, _, _, dg, off, ln = params(st, ubase, ibase)
        return pltpu.async_copy(
            src.at[pl.ds(dg * 8, 8), pl.ds(off, ln)],
            buf.at[pl.ds(0, 8), pl.ds(0, ln)], sem_in)

    def issue_outs(st, ubase, ibase, buf):
        _, dst, n_eff, dg, off, ln = params(st, ubase, ibase)
        cps = []
        for dd in range(8):
            d = dg * 8 + dd
            cps.append(pltpu.async_copy(
                buf.at[dd, pl.ds(0, ln)],
                dst.at[pl.ds(d * n_eff + off, ln)], sem_out))
        return cps

    ubase = wid * U_SPAN
    ibase = wid * I_SPAN
    ins = [None, None]
    outs = [None, None]
    ins[0] = issue_in(steps[0], ubase, ibase, slabs[0])
    for s in range(len(steps)):
        p = s % 2
        if s + 1 < len(steps):
            if outs[1 - p] is not None:
                for cp in outs[1 - p]:
                    cp.wait()
            ins[1 - p] = issue_in(steps[s + 1], ubase, ibase, slabs[1 - p])
        ins[p].wait()
        outs[p] = issue_outs(steps[s], ubase, ibase, slabs[p])
    for cps in outs:
        if cps is not None:
            for cp in cps:
                cp.wait()

    @pl.when(is_last)
    def _():
        for dg in range(DGROUPS):
            for st in (("u", dg, 0, U_REM), ("i", dg, 0, I_REM)):
                issue_in(st, U_REM_OFF, I_REM_OFF, slabs[0]).wait()
                for cp in issue_outs(st, U_REM_OFF, I_REM_OFF, slabs[0]):
                    cp.wait()

    @pl.when(wid == 0)
    def _():
        cu = pltpu.async_copy(uapp_hbm, uapp_v, sem_in)
        ci = pltpu.async_copy(iapp_hbm, iapp_v, sem_in)
        cu.wait()
        ci.wait()
        cps = []
        for d in range(EMB_DIM):
            cps.append(pltpu.async_copy(
                uapp_v.at[d],
                uflat_hbm.at[pl.ds(U_APP_OFF + d * U_APP, U_APP)], sem_out))
            cps.append(pltpu.async_copy(
                iapp_v.at[d],
                iflat_hbm.at[pl.ds(I_APP_OFF + d * I_APP, I_APP)], sem_out))
        for cp in cps:
            cp.wait()


def _dot_kernel(uidx_hbm, iidx_hbm, uflat_hbm, iflat_hbm, out_hbm,
                uidx_v, iidx_v, ubuf, ibuf, out_v, sem_s, sem_u, sem_i):
    wid = lax.axis_index("s") * NUM_CORES + lax.axis_index("c")

    cs_u = pltpu.async_copy(uidx_hbm.at[wid], uidx_v, sem_s)
    cs_i = pltpu.async_copy(iidx_hbm.at[wid], iidx_v, sem_s)
    cs_u.wait()
    cs_i.wait()

    cu = pltpu.async_copy(uflat_hbm.at[uidx_v], ubuf, sem_u)
    ci = pltpu.async_copy(iflat_hbm.at[iidx_v], ibuf, sem_i)
    cu.wait()
    ci.wait()

    @pl.loop(0, BLOCKS)
    def _(j):
        acc = jnp.zeros((LANES,), jnp.float32)
        for d in range(EMB_DIM):
            acc = acc + (ubuf[pl.ds(d * B_PER_W + j * LANES, LANES)]
                         * ibuf[pl.ds(d * B_PER_W + j * LANES, LANES)])
        out_v[pl.ds(j * LANES, LANES)] = acc

    pltpu.sync_copy(out_v, out_hbm.at[pl.ds(wid * B_PER_W, B_PER_W)])


@jax.jit
def kernel(user, item, user_emb, item_emb):
    user = user.astype(jnp.int32)
    item = item.astype(jnp.int32)
    dim_off = lax.iota(jnp.int32, EMB_DIM)

    u_b = user.reshape(NUM_WORKERS, 1, B_PER_W)
    i_b = item.reshape(NUM_WORKERS, 1, B_PER_W)
    d_b = dim_off[None, :, None]
    uidx = jnp.where(u_b < U_EFF,
                     d_b * U_EFF + u_b,
                     U_APP_OFF + d_b * U_APP + (u_b - U_EFF)
                     ).reshape(NUM_WORKERS, SLAB)
    iidx = jnp.where(i_b < I_EFF,
                     d_b * I_EFF + i_b,
                     I_APP_OFF + d_b * I_APP + (i_b - I_EFF)
                     ).reshape(NUM_WORKERS, SLAB)

    uapp = user_emb[U_EFF:, :].T
    iapp = item_emb[I_EFF:, :].T

    mesh = plsc.VectorSubcoreMesh(core_axis_name="c", subcore_axis_name="s")
    relayout = pl.kernel(
        _relayout_kernel,
        out_type=(jax.ShapeDtypeStruct((EMB_DIM * N_USER_ROWS,), jnp.float32),
                  jax.ShapeDtypeStruct((EMB_DIM * N_ITEM_ROWS,), jnp.float32)),
        mesh=mesh,
        compiler_params=pltpu.CompilerParams(needs_layout_passes=False),
        scratch_types=[
            pltpu.VMEM((8, SLAB_W), jnp.float32),
            pltpu.VMEM((8, SLAB_W), jnp.float32),
            pltpu.VMEM((EMB_DIM, U_APP), jnp.float32),
            pltpu.VMEM((EMB_DIM, I_APP), jnp.float32),
            pltpu.SemaphoreType.DMA,
            pltpu.SemaphoreType.DMA,
        ],
    )
    uflat, iflat = relayout(user_emb.T, item_emb.T, uapp, iapp)

    gather = pl.kernel(
        _dot_kernel,
        out_type=jax.ShapeDtypeStruct((BATCH,), jnp.float32),
        mesh=mesh,
        compiler_params=pltpu.CompilerParams(needs_layout_passes=False),
        scratch_types=[
            pltpu.VMEM((SLAB,), jnp.int32),
            pltpu.VMEM((SLAB,), jnp.int32),
            pltpu.VMEM((SLAB,), jnp.float32),
            pltpu.VMEM((SLAB,), jnp.float32),
            pltpu.VMEM((B_PER_W,), jnp.float32),
            pltpu.SemaphoreType.DMA,
            pltpu.SemaphoreType.DMA,
            pltpu.SemaphoreType.DMA,
        ],
    )
    return gather(uidx, iidx, uflat, iflat)


# speedup vs baseline: 14.6369x; 1.0002x over previous
import jax
import jax.numpy as jnp
from jax import lax
from jax.experimental import pallas as pl
from jax.experimental.pallas import tpu as pltpu
from jax.experimental.pallas import tpu_sc as plsc

BATCH = 16384
EMB_DIM = 32
N_USER_ROWS = 1000000
N_ITEM_ROWS = 100000
NUM_CORES = 2
NUM_SUBCORES = 16
NUM_WORKERS = NUM_CORES * NUM_SUBCORES
B_PER_W = BATCH // NUM_WORKERS
LANES = 16
BLOCKS = B_PER_W // LANES
SLAB = EMB_DIM * B_PER_W
DGROUPS = EMB_DIM // 8

U_EFF = 999936
U_APP = N_USER_ROWS - U_EFF
U_SPAN = 31232
U_REM_OFF = NUM_WORKERS * U_SPAN
U_REM = U_EFF - U_REM_OFF
SLAB_W = 3968
U_CHUNKS = [3968] * 7 + [3456]
U_APP_OFF = EMB_DIM * U_EFF

I_EFF = 99968
I_APP = N_ITEM_ROWS - I_EFF
I_SPAN = 3072
I_REM_OFF = NUM_WORKERS * I_SPAN
I_REM = I_EFF - I_REM_OFF
I_CHUNKS = [3072]
I_APP_OFF = EMB_DIM * I_EFF


def _relayout_kernel(uT_hbm, iT_hbm, uapp_hbm, iapp_hbm,
                     uflat_hbm, iflat_hbm,
                     slab0, slab1, uapp_v, iapp_v,
                     sem_in0, sem_in1, sem_out0, sem_out1):
    wid = lax.axis_index("s") * NUM_CORES + lax.axis_index("c")
    is_last = wid >= NUM_WORKERS - 1
    slabs = (slab0, slab1)

    steps = []
    for dg in range(DGROUPS):
        coff = 0
        for ln in U_CHUNKS:
            steps.append(("u", dg, coff, ln))
            coff += ln
        steps.append(("i", dg, 0, I_SPAN))

    def params(st, ubase, ibase):
        tab, dg, coff, ln = st
        if tab == "u":
            return uT_hbm, uflat_hbm, U_EFF, dg, ubase + coff, ln
        return iT_hbm, iflat_hbm, I_EFF, dg, ibase + coff, ln

    ins_sems = (sem_in0, sem_in1)
    out_sems = (sem_out0, sem_out1)

    def issue_in(st, ubase, ibase, buf, sem):
        src, _, _, dg, off, ln = params(st, ubase, ibase)
        return pltpu.async_copy(
            src.at[pl.ds(dg * 8, 8), pl.ds(off, ln)],
            buf.at[pl.ds(0, 8), pl.ds(0, ln)], sem)

    def issue_outs(st, ubase, ibase, buf, sem):
        _, dst, n_eff, dg, off, ln = params(st, ubase, ibase)
        cps = []
        for dd in range(8):
            d = dg * 8 + dd
            cps.append(pltpu.async_copy(
                buf.at[dd, pl.ds(0, ln)],
                dst.at[pl.ds(d * n_eff + off, ln)], sem))
        return cps

    ubase = wid * U_SPAN
    ibase = wid * I_SPAN
    ins = [None, None]
    outs = [None, None]
    ins[0] = issue_in(steps[0], ubase, ibase, slabs[0], ins_sems[0])
    for s in range(len(steps)):
        p = s % 2
        if s + 1 < len(steps):
            if outs[1 - p] is not None:
                for cp in outs[1 - p]:
                    cp.wait()
            ins[1 - p] = issue_in(steps[s + 1], ubase, ibase,
                                  slabs[1 - p], ins_sems[1 - p])
        ins[p].wait()
        outs[p] = issue_outs(steps[s], ubase, ibase, slabs[p], out_sems[p])
    for cps in outs:
        if cps is not None:
            for cp in cps:
                cp.wait()

    @pl.when(is_last)
    def _():
        for dg in range(DGROUPS):
            for st in (("u", dg, 0, U_REM), ("i", dg, 0, I_REM)):
                issue_in(st, U_REM_OFF, I_REM_OFF, slabs[0],
                         ins_sems[0]).wait()
                for cp in issue_outs(st, U_REM_OFF, I_REM_OFF, slabs[0],
                                     out_sems[0]):
                    cp.wait()

    @pl.when(wid == 0)
    def _():
        cu = pltpu.async_copy(uapp_hbm, uapp_v, sem_in0)
        ci = pltpu.async_copy(iapp_hbm, iapp_v, sem_in0)
        cu.wait()
        ci.wait()
        cps = []
        for d in range(EMB_DIM):
            cps.append(pltpu.async_copy(
                uapp_v.at[d],
                uflat_hbm.at[pl.ds(U_APP_OFF + d * U_APP, U_APP)], sem_out0))
            cps.append(pltpu.async_copy(
                iapp_v.at[d],
                iflat_hbm.at[pl.ds(I_APP_OFF + d * I_APP, I_APP)], sem_out0))
        for cp in cps:
            cp.wait()


def _dot_kernel(uidx_hbm, iidx_hbm, uflat_hbm, iflat_hbm, out_hbm,
                uidx_v, iidx_v, ubuf, ibuf, out_v, sem_s, sem_u, sem_i):
    wid = lax.axis_index("s") * NUM_CORES + lax.axis_index("c")

    cs_u = pltpu.async_copy(uidx_hbm.at[wid], uidx_v, sem_s)
    cs_i = pltpu.async_copy(iidx_hbm.at[wid], iidx_v, sem_s)
    cs_u.wait()
    cs_i.wait()

    cu = pltpu.async_copy(uflat_hbm.at[uidx_v], ubuf, sem_u)
    ci = pltpu.async_copy(iflat_hbm.at[iidx_v], ibuf, sem_i)
    cu.wait()
    ci.wait()

    @pl.loop(0, BLOCKS)
    def _(j):
        acc = jnp.zeros((LANES,), jnp.float32)
        for d in range(EMB_DIM):
            acc = acc + (ubuf[pl.ds(d * B_PER_W + j * LANES, LANES)]
                         * ibuf[pl.ds(d * B_PER_W + j * LANES, LANES)])
        out_v[pl.ds(j * LANES, LANES)] = acc

    pltpu.sync_copy(out_v, out_hbm.at[pl.ds(wid * B_PER_W, B_PER_W)])


@jax.jit
def kernel(user, item, user_emb, item_emb):
    user = user.astype(jnp.int32)
    item = item.astype(jnp.int32)
    dim_off = lax.iota(jnp.int32, EMB_DIM)

    u_b = user.reshape(NUM_WORKERS, 1, B_PER_W)
    i_b = item.reshape(NUM_WORKERS, 1, B_PER_W)
    d_b = dim_off[None, :, None]
    uidx = jnp.where(u_b < U_EFF,
                     d_b * U_EFF + u_b,
                     U_APP_OFF + d_b * U_APP + (u_b - U_EFF)
                     ).reshape(NUM_WORKERS, SLAB)
    iidx = jnp.where(i_b < I_EFF,
                     d_b * I_EFF + i_b,
                     I_APP_OFF + d_b * I_APP + (i_b - I_EFF)
                     ).reshape(NUM_WORKERS, SLAB)

    uapp = user_emb[U_EFF:, :].T
    iapp = item_emb[I_EFF:, :].T

    mesh = plsc.VectorSubcoreMesh(core_axis_name="c", subcore_axis_name="s")
    relayout = pl.kernel(
        _relayout_kernel,
        out_type=(jax.ShapeDtypeStruct((EMB_DIM * N_USER_ROWS,), jnp.float32),
                  jax.ShapeDtypeStruct((EMB_DIM * N_ITEM_ROWS,), jnp.float32)),
        mesh=mesh,
        compiler_params=pltpu.CompilerParams(needs_layout_passes=False),
        scratch_types=[
            pltpu.VMEM((8, SLAB_W), jnp.float32),
            pltpu.VMEM((8, SLAB_W), jnp.float32),
            pltpu.VMEM((EMB_DIM, U_APP), jnp.float32),
            pltpu.VMEM((EMB_DIM, I_APP), jnp.float32),
            pltpu.SemaphoreType.DMA,
            pltpu.SemaphoreType.DMA,
            pltpu.SemaphoreType.DMA,
            pltpu.SemaphoreType.DMA,
        ],
    )
    uflat, iflat = relayout(user_emb.T, item_emb.T, uapp, iapp)

    gather = pl.kernel(
        _dot_kernel,
        out_type=jax.ShapeDtypeStruct((BATCH,), jnp.float32),
        mesh=mesh,
        compiler_params=pltpu.CompilerParams(needs_layout_passes=False),
        scratch_types=[
            pltpu.VMEM((SLAB,), jnp.int32),
            pltpu.VMEM((SLAB,), jnp.int32),
            pltpu.VMEM((SLAB,), jnp.float32),
            pltpu.VMEM((SLAB,), jnp.float32),
            pltpu.VMEM((B_PER_W,), jnp.float32),
            pltpu.SemaphoreType.DMA,
            pltpu.SemaphoreType.DMA,
            pltpu.SemaphoreType.DMA,
        ],
    )
    return gather(uidx, iidx, uflat, iflat)
